# baseline (speedup 1.0000x reference)
import functools

import jax
import jax.numpy as jnp
from jax import lax
from jax.experimental import pallas as pl
from jax.experimental.pallas import tpu as pltpu
from jax.experimental.pallas import tpu_sc as plsc

BATCH = 16384
EMB = 64
NC = 2
NS = 16
NW = NC * NS
B_PER_W = BATCH // NW
CHUNK = 128
NCHUNK = B_PER_W // CHUNK

TR_W = 512
TR_NBLK = -(-1000000 // TR_W)
NROW2 = TR_NBLK * (TR_W // 2)

_SHUF_DNUMS = lax.GatherDimensionNumbers(
    offset_dims=(), collapsed_slice_dims=(0,), start_index_map=(0,))


def _shuffle(x, perm):
    return lax.gather(x, perm[:, None], _SHUF_DNUMS, slice_sizes=(1,),
                      mode=lax.GatherScatterMode.PROMISE_IN_BOUNDS)


def _tr_body(x_ref, o_ref):
    y = jnp.transpose(x_ref[...])
    o_ref[:, 0:EMB] = y[0:TR_W // 2]
    o_ref[:, EMB:2 * EMB] = y[TR_W // 2:TR_W]


def _transpose_pack(tbl_T):
    return pl.pallas_call(
        _tr_body,
        grid=(TR_NBLK,),
        in_specs=[pl.BlockSpec((EMB, TR_W), lambda j: (0, j))],
        out_specs=pl.BlockSpec((TR_W // 2, 2 * EMB), lambda j: (j, 0)),
        out_shape=jax.ShapeDtypeStruct((NROW2, 2 * EMB), jnp.float32),
    )(tbl_T)


def _body(u_hbm, v_hbm, user_hbm, item_hbm, out_hbm,
          u_raw, v_raw, u_idx, v_idx, ue, ve, out_v, sem0, sem1):
    wid = lax.axis_index("s") * NC + lax.axis_index("c")
    base = wid * B_PER_W

    for j in range(NCHUNK):
        pltpu.sync_copy(u_hbm.at[pl.ds(base + j * CHUNK, CHUNK)], u_raw.at[j])
        pltpu.sync_copy(v_hbm.at[pl.ds(base + j * CHUNK, CHUNK)], v_raw.at[j])
    for j in range(NCHUNK):
        for t in range(CHUNK // 16):
            sl = pl.ds(t * 16, 16)
            uv = u_raw[j, sl]
            u_idx[j, sl] = (lax.shift_right_logical(uv, 9) * 256 +
                            (uv & 255))
            v_idx[j, sl] = lax.shift_right_logical(v_raw[j, sl], 1)

    sems = (sem0, sem1)

    def fire(c):
        b = c % 2
        return (pltpu.async_copy(user_hbm.at[u_idx.at[c]], ue.at[b], sems[b]),
                pltpu.async_copy(item_hbm.at[v_idx.at[c]], ve.at[b], sems[b]))

    lanes = lax.iota(jnp.int32, 16)
    zero16 = jnp.zeros((16,), jnp.float32)

    inflight = fire(0)
    for c in range(NCHUNK):
        nxt = fire(c + 1) if c + 1 < NCHUNK else None
        for cp in inflight:
            cp.wait()
        inflight = nxt
        b = c % 2

        def group_body(g, carry, c=c, b=b):
            r0 = g * 16
            acc = zero16
            pu_vec = (lax.shift_right_logical(u_raw[c, pl.ds(r0, 16)], 8)
                      & 1) * EMB
            pv_vec = (v_raw[c, pl.ds(r0, 16)] & 1) * EMB
            for k in range(16):
                r = r0 + k
                pu = pu_vec[k]
                pv = pv_vec[k]
                p = ue[b, r, pl.ds(pu, 16)] * ve[b, r, pl.ds(pv, 16)]
                for q in range(1, EMB // 16):
                    p = p + (ue[b, r, pl.ds(pu + q * 16, 16)] *
                             ve[b, r, pl.ds(pv + q * 16, 16)])
                for s in (8, 4, 2, 1):
                    p = p + _shuffle(p, lanes ^ s)
                acc = jnp.where(lanes == k, p, acc)
            out_v[pl.ds(c * CHUNK + r0, 16)] = acc
            return carry

        lax.fori_loop(0, CHUNK // 16, group_body, 0)

    pltpu.sync_copy(out_v, out_hbm.at[pl.ds(base, B_PER_W)])


@jax.jit
def _run(u, v, user_emb, item_emb):
    mesh = plsc.VectorSubcoreMesh(core_axis_name="c", subcore_axis_name="s")
    kfn = functools.partial(
        pl.kernel,
        mesh=mesh,
        out_type=jax.ShapeDtypeStruct((BATCH,), jnp.float32),
        scratch_types=[
            pltpu.VMEM((NCHUNK, CHUNK), jnp.int32),
            pltpu.VMEM((NCHUNK, CHUNK), jnp.int32),
            pltpu.VMEM((NCHUNK, CHUNK), jnp.int32),
            pltpu.VMEM((NCHUNK, CHUNK), jnp.int32),
            pltpu.VMEM((2, CHUNK, 2 * EMB), jnp.float32),
            pltpu.VMEM((2, CHUNK, 2 * EMB), jnp.float32),
            pltpu.VMEM((B_PER_W,), jnp.float32),
            pltpu.SemaphoreType.DMA,
            pltpu.SemaphoreType.DMA,
        ],
    )(_body)
    user2 = _transpose_pack(user_emb.T)
    item2 = item_emb.reshape(-1, 2 * EMB)
    return kfn(u, v, user2, item2)


def kernel(u, v, user_emb, item_emb):
    return _run(u, v, user_emb, item_emb)


# speedup vs baseline: 2.0564x; 2.0564x over previous
import functools

import jax
import jax.numpy as jnp
from jax import lax
from jax.experimental import pallas as pl
from jax.experimental.pallas import tpu as pltpu
from jax.experimental.pallas import tpu_sc as plsc

BATCH = 16384
EMB = 64
NC = 2
NS = 16
NW = NC * NS
B_PER_W = BATCH // NW
CHUNK = 128
NCHUNK = B_PER_W // CHUNK

TR_W = 4096
TR_SH = TR_W.bit_length() - 1
TR_NBLK = -(-1000000 // TR_W)
NROW2 = TR_NBLK * (TR_W // 2)

_SHUF_DNUMS = lax.GatherDimensionNumbers(
    offset_dims=(), collapsed_slice_dims=(0,), start_index_map=(0,))


def _shuffle(x, perm):
    return lax.gather(x, perm[:, None], _SHUF_DNUMS, slice_sizes=(1,),
                      mode=lax.GatherScatterMode.PROMISE_IN_BOUNDS)


def _tr_body(x_ref, o_ref):
    y = jnp.transpose(x_ref[...])
    o_ref[:, 0:EMB] = y[0:TR_W // 2]
    o_ref[:, EMB:2 * EMB] = y[TR_W // 2:TR_W]


def _transpose_pack(tbl_T):
    return pl.pallas_call(
        _tr_body,
        grid=(TR_NBLK,),
        in_specs=[pl.BlockSpec((EMB, TR_W), lambda j: (0, j))],
        out_specs=pl.BlockSpec((TR_W // 2, 2 * EMB), lambda j: (j, 0)),
        out_shape=jax.ShapeDtypeStruct((NROW2, 2 * EMB), jnp.float32),
    )(tbl_T)


def _body(u_hbm, v_hbm, user_hbm, item_hbm, out_hbm,
          u_raw, v_raw, u_idx, v_idx, ue, ve, out_v, sem0, sem1):
    wid = lax.axis_index("s") * NC + lax.axis_index("c")
    base = wid * B_PER_W

    for j in range(NCHUNK):
        pltpu.sync_copy(u_hbm.at[pl.ds(base + j * CHUNK, CHUNK)], u_raw.at[j])
        pltpu.sync_copy(v_hbm.at[pl.ds(base + j * CHUNK, CHUNK)], v_raw.at[j])
    for j in range(NCHUNK):
        for t in range(CHUNK // 16):
            sl = pl.ds(t * 16, 16)
            uv = u_raw[j, sl]
            u_idx[j, sl] = (lax.shift_right_logical(uv, TR_SH) * (TR_W // 2) +
                            (uv & (TR_W // 2 - 1)))
            v_idx[j, sl] = lax.shift_right_logical(v_raw[j, sl], 1)

    sems = (sem0, sem1)

    def fire(c):
        b = c % 2
        return (pltpu.async_copy(user_hbm.at[u_idx.at[c]], ue.at[b], sems[b]),
                pltpu.async_copy(item_hbm.at[v_idx.at[c]], ve.at[b], sems[b]))

    lanes = lax.iota(jnp.int32, 16)
    zero16 = jnp.zeros((16,), jnp.float32)

    inflight = fire(0)
    for c in range(NCHUNK):
        nxt = fire(c + 1) if c + 1 < NCHUNK else None
        for cp in inflight:
            cp.wait()
        inflight = nxt
        b = c % 2

        def group_body(g, carry, c=c, b=b):
            r0 = g * 16
            acc = zero16
            pu_vec = (lax.shift_right_logical(u_raw[c, pl.ds(r0, 16)],
                                              TR_SH - 1) & 1) * EMB
            pv_vec = (v_raw[c, pl.ds(r0, 16)] & 1) * EMB
            for k in range(16):
                r = r0 + k
                pu = pu_vec[k]
                pv = pv_vec[k]
                p = ue[b, r, pl.ds(pu, 16)] * ve[b, r, pl.ds(pv, 16)]
                for q in range(1, EMB // 16):
                    p = p + (ue[b, r, pl.ds(pu + q * 16, 16)] *
                             ve[b, r, pl.ds(pv + q * 16, 16)])
                for s in (8, 4, 2, 1):
                    p = p + _shuffle(p, lanes ^ s)
                acc = jnp.where(lanes == k, p, acc)
            out_v[pl.ds(c * CHUNK + r0, 16)] = acc
            return carry

        lax.fori_loop(0, CHUNK // 16, group_body, 0)

    pltpu.sync_copy(out_v, out_hbm.at[pl.ds(base, B_PER_W)])


@jax.jit
def _run(u, v, user_emb, item_emb):
    mesh = plsc.VectorSubcoreMesh(core_axis_name="c", subcore_axis_name="s")
    kfn = functools.partial(
        pl.kernel,
        mesh=mesh,
        out_type=jax.ShapeDtypeStruct((BATCH,), jnp.float32),
        scratch_types=[
            pltpu.VMEM((NCHUNK, CHUNK), jnp.int32),
            pltpu.VMEM((NCHUNK, CHUNK), jnp.int32),
            pltpu.VMEM((NCHUNK, CHUNK), jnp.int32),
            pltpu.VMEM((NCHUNK, CHUNK), jnp.int32),
            pltpu.VMEM((2, CHUNK, 2 * EMB), jnp.float32),
            pltpu.VMEM((2, CHUNK, 2 * EMB), jnp.float32),
            pltpu.VMEM((B_PER_W,), jnp.float32),
            pltpu.SemaphoreType.DMA,
            pltpu.SemaphoreType.DMA,
        ],
    )(_body)
    user2 = _transpose_pack(user_emb.T)
    item2 = item_emb.reshape(-1, 2 * EMB)
    return kfn(u, v, user2, item2)


def kernel(u, v, user_emb, item_emb):
    return _run(u, v, user_emb, item_emb)
